# Initial kernel scaffold; baseline (speedup 1.0000x reference)
import jax
import jax.numpy as jnp
from jax.experimental import pallas as pl


def kernel(input_ids, segment_ids, token_embedding_matrix, segment_embedding_matrix, position_embedding_matrix):
    raise NotImplementedError("write your pallas kernel here")


# speedup vs baseline: 1.7394x; 1.7394x over previous
import functools

import jax
import jax.numpy as jnp
from jax import lax
from jax.experimental import pallas as pl
from jax.experimental.pallas import tpu as pltpu
from jax.experimental.pallas import tpu_sc as plsc

VOCAB = 1000000
D = 64
NSEG = 16
SEQ = 512
NC = 2
NS = 16
NW = NC * NS
CHUNK = 128
LANES = 16


def _body(ids_hbm, segs_hbm, tok_hbm, seg_hbm, pos_hbm,
          out_hbm, comb_hbm,
          build_v, segrow_v, idx_tok_v, seg_v, idx_comb_v, tok_v, comb_v,
          sem_a, sem_b):
    c = lax.axis_index("c")
    s = lax.axis_index("s")
    wid = c * NS + s
    n_tokens = out_hbm.shape[0]
    per_w = n_tokens // NW
    core_off = c * (NSEG * SEQ)

    pltpu.sync_copy(pos_hbm, build_v)
    pltpu.sync_copy(seg_hbm.at[s], segrow_v)

    def build_row(r, _):
        for j in range(D // LANES):
            sl = pl.ds(j * LANES, LANES)
            build_v[r, sl] = build_v[r, sl] + segrow_v[sl]
        return _

    lax.fori_loop(0, SEQ, build_row, 0)
    pltpu.sync_copy(build_v, comb_hbm.at[pl.ds(core_off + s * SEQ, SEQ)])
    plsc.subcore_barrier()

    wbase = wid * per_w
    iota = lax.iota(jnp.int32, LANES)

    def chunk_step(ci, _):
        base = wbase + ci * CHUNK
        pos_base = lax.rem(base, SEQ)
        pltpu.sync_copy(ids_hbm.at[pl.ds(base, CHUNK)], idx_tok_v)
        pltpu.sync_copy(segs_hbm.at[pl.ds(base, CHUNK)], seg_v)
        for g in range(CHUNK // LANES):
            sl = pl.ds(g * LANES, LANES)
            idx_comb_v[sl] = (seg_v[sl] * SEQ
                              + (pos_base + g * LANES + core_off) + iota)
        cp_tok = pltpu.async_copy(tok_hbm.at[idx_tok_v], tok_v, sem_a)
        cp_comb = pltpu.async_copy(comb_hbm.at[idx_comb_v], comb_v, sem_b)
        cp_tok.wait()
        cp_comb.wait()

        def add_row(r, __):
            for j in range(D // LANES):
                sl = pl.ds(j * LANES, LANES)
                tok_v[r, sl] = tok_v[r, sl] + comb_v[r, sl]
            return __

        lax.fori_loop(0, CHUNK, add_row, 0)
        pltpu.sync_copy(tok_v, out_hbm.at[pl.ds(base, CHUNK)])
        return _

    lax.fori_loop(0, per_w // CHUNK, chunk_step, 0)


def kernel(input_ids, segment_ids, token_embedding_matrix,
           segment_embedding_matrix, position_embedding_matrix):
    batch, seq = input_ids.shape
    n = batch * seq
    ids = input_ids.reshape(n).astype(jnp.int32)
    segs = segment_ids.reshape(n).astype(jnp.int32)

    mesh = plsc.VectorSubcoreMesh(core_axis_name="c", subcore_axis_name="s",
                                  num_cores=NC, num_subcores=NS)
    run = pl.kernel(
        _body,
        out_type=(
            jax.ShapeDtypeStruct((n, D), jnp.float32),
            jax.ShapeDtypeStruct((NC * NSEG * SEQ, D), jnp.float32),
        ),
        mesh=mesh,
        compiler_params=pltpu.CompilerParams(use_tc_tiling_on_sc=False),
        scratch_types=(
            pltpu.VMEM((SEQ, D), jnp.float32),
            pltpu.VMEM((D,), jnp.float32),
            pltpu.VMEM((CHUNK,), jnp.int32),
            pltpu.VMEM((CHUNK,), jnp.int32),
            pltpu.VMEM((CHUNK,), jnp.int32),
            pltpu.VMEM((CHUNK, D), jnp.float32),
            pltpu.VMEM((CHUNK, D), jnp.float32),
            pltpu.SemaphoreType.DMA,
            pltpu.SemaphoreType.DMA,
        ),
    )
    out, _ = run(ids, segs,
                 token_embedding_matrix.astype(jnp.float32),
                 segment_embedding_matrix.astype(jnp.float32),
                 position_embedding_matrix.astype(jnp.float32))
    return out.reshape(batch, seq, D)


# speedup vs baseline: 2.0153x; 1.1586x over previous
import jax
import jax.numpy as jnp
from jax import lax
from jax.experimental import pallas as pl
from jax.experimental.pallas import tpu as pltpu
from jax.experimental.pallas import tpu_sc as plsc

D = 64
NSEG = 16
SEQ = 512
NC = 2
NS = 16
NW = NC * NS
CHUNK = 256
NSTREAM = CHUNK // 128
LANES = 16
NBUF = 2


def _body(ids_hbm, segs_hbm, tok_hbm, seg_hbm, pos_hbm,
          out_hbm, comb_hbm,
          build_v, segrow_v,
          idx_tok_v, idx_comb_v, seg_v, tok_v, comb_v,
          sem_tok, sem_comb, sem_out):
    c = lax.axis_index("c")
    s = lax.axis_index("s")
    wid = c * NS + s
    batch, seq = ids_hbm.shape
    n_tokens = batch * seq
    per_w = n_tokens // NW
    nchunk = per_w // CHUNK
    core_off = c * (NSEG * SEQ)

    pltpu.sync_copy(pos_hbm, build_v)
    pltpu.sync_copy(seg_hbm.at[s], segrow_v)

    def build_row(r, carry):
        for j in range(D // LANES):
            sl = pl.ds(j * LANES, LANES)
            build_v[r, sl] = build_v[r, sl] + segrow_v[sl]
        return carry

    lax.fori_loop(0, SEQ, build_row, 0)
    pltpu.sync_copy(build_v, comb_hbm.at[pl.ds(core_off + s * SEQ, SEQ)])
    plsc.subcore_barrier()

    wbase = wid * per_w
    iota = lax.iota(jnp.int32, LANES)

    def gather_descs(b):
        descs = []
        for j in range(NSTREAM):
            dst_sl = pl.ds(j * 128, 128)
            descs.append(pltpu.make_async_copy(
                tok_hbm.at[idx_tok_v[b].at[j]], tok_v[b].at[dst_sl],
                sem_tok[b]))
            descs.append(pltpu.make_async_copy(
                comb_hbm.at[idx_comb_v[b].at[j]], comb_v[b].at[dst_sl],
                sem_comb[b]))
        return descs

    def out_slot(i):
        base = wbase + i * CHUNK
        return out_hbm.at[base // SEQ, pl.ds(lax.rem(base, SEQ), CHUNK)]

    def start(i, b):
        base = wbase + i * CHUNK
        row = base // SEQ
        col = lax.rem(base, SEQ)

        @pl.when(i >= NBUF)
        def _():
            pltpu.make_async_copy(tok_v[b], out_slot(i), sem_out[b]).wait()

        for j in range(NSTREAM):
            pltpu.sync_copy(ids_hbm.at[row, pl.ds(col + j * 128, 128)],
                            idx_tok_v[b].at[j])
        pltpu.sync_copy(segs_hbm.at[row, pl.ds(col, CHUNK)], seg_v[b])
        for g in range(CHUNK // LANES):
            j, off = divmod(g * LANES, 128)
            idx_comb_v[b][j, pl.ds(off, LANES)] = (
                seg_v[b][pl.ds(g * LANES, LANES)] * SEQ
                + (col + g * LANES + core_off) + iota)
        for d in gather_descs(b):
            d.start()

    def finish(i, b):
        for d in gather_descs(b):
            d.wait()

        def add_row(r, carry):
            for j in range(D // LANES):
                sl = pl.ds(j * LANES, LANES)
                tok_v[b][r, sl] = tok_v[b][r, sl] + comb_v[b][r, sl]
            return carry

        lax.fori_loop(0, CHUNK, add_row, 0)
        pltpu.async_copy(tok_v[b], out_slot(i), sem_out[b])

    for b in range(NBUF):
        start(b, b)

    def pair_step(g, carry):
        for b in range(NBUF):
            i = g * NBUF + b
            finish(i, b)

            @pl.when(i + NBUF < nchunk)
            def _():
                start(i + NBUF, b)
        return carry

    lax.fori_loop(0, nchunk // NBUF, pair_step, 0)
    for b in range(NBUF):
        pltpu.make_async_copy(tok_v[b], out_slot(0), sem_out[b]).wait()


def kernel(input_ids, segment_ids, token_embedding_matrix,
           segment_embedding_matrix, position_embedding_matrix):
    batch, seq = input_ids.shape

    mesh = plsc.VectorSubcoreMesh(core_axis_name="c", subcore_axis_name="s",
                                  num_cores=NC, num_subcores=NS)
    run = pl.kernel(
        _body,
        out_type=(
            jax.ShapeDtypeStruct((batch, seq, D), jnp.float32),
            jax.ShapeDtypeStruct((NC * NSEG * SEQ, D), jnp.float32),
        ),
        mesh=mesh,
        compiler_params=pltpu.CompilerParams(use_tc_tiling_on_sc=False),
        scratch_types=(
            pltpu.VMEM((SEQ, D), jnp.float32),
            pltpu.VMEM((D,), jnp.float32),
            [pltpu.VMEM((NSTREAM, 128), jnp.int32)] * NBUF,
            [pltpu.VMEM((NSTREAM, 128), jnp.int32)] * NBUF,
            [pltpu.VMEM((CHUNK,), jnp.int32)] * NBUF,
            [pltpu.VMEM((CHUNK, D), jnp.float32)] * NBUF,
            [pltpu.VMEM((CHUNK, D), jnp.float32)] * NBUF,
            [pltpu.SemaphoreType.DMA] * NBUF,
            [pltpu.SemaphoreType.DMA] * NBUF,
            [pltpu.SemaphoreType.DMA] * NBUF,
        ),
    )
    out, _ = run(input_ids.astype(jnp.int32), segment_ids.astype(jnp.int32),
                 token_embedding_matrix.astype(jnp.float32),
                 segment_embedding_matrix.astype(jnp.float32),
                 position_embedding_matrix.astype(jnp.float32))
    return out
